# baseline (speedup 1.0000x reference)
import jax
import jax.numpy as jnp
from jax import lax
from jax.experimental import pallas as pl
from jax.experimental.pallas import tpu as pltpu

N = 4096
NFEAT = 256
NHID = 128
TAU = 0.5
BLK = 256
NB = N // BLK


def _mega_body(x1_ref, x2_ref, w11_ref, w21_ref, w12_ref, w22_ref,
               b11_ref, b21_ref, b12_ref, b22_ref,
               adj1_ref, adj2_ref, clm_ref,
               z1_ref, z2_ref, loss_ref,
               s21_ref, s22_ref, zn1_ref, zn2_ref):
    i = pl.program_id(0)

    def gcn_mid(adj_ref, x_ref, w1_ref, b1_ref, w2_ref, s2_ref, k):
        t = jnp.dot(adj_ref[...].astype(jnp.bfloat16), x_ref[...],
                    preferred_element_type=jnp.float32)
        t = jnp.dot(t.astype(jnp.bfloat16), w1_ref[...],
                    preferred_element_type=jnp.float32)
        h = jnp.maximum(t + b1_ref[...], 0.0).astype(jnp.bfloat16)
        s2_ref[pl.ds(k * BLK, BLK), :] = jnp.dot(
            h, w2_ref[...], preferred_element_type=jnp.float32
        ).astype(jnp.bfloat16)

    def gcn_out(adj_ref, s2_ref, b2_ref, z_ref, zn_ref, k):
        z = jnp.dot(adj_ref[...].astype(jnp.bfloat16), s2_ref[...],
                    preferred_element_type=jnp.float32) + b2_ref[...]
        z_ref[...] = z
        nrm = jnp.sqrt(jnp.sum(z * z, axis=1, keepdims=True))
        zn_ref[pl.ds(k * BLK, BLK), :] = (z / nrm).astype(jnp.bfloat16)

    @pl.when(i < NB)
    def _():
        gcn_mid(adj1_ref, x1_ref, w11_ref, b11_ref, w12_ref, s21_ref, i)

    @pl.when((NB <= i) & (i < 2 * NB))
    def _():
        gcn_mid(adj2_ref, x2_ref, w21_ref, b21_ref, w22_ref, s22_ref, i - NB)

    @pl.when((2 * NB <= i) & (i < 3 * NB))
    def _():
        gcn_out(adj1_ref, s21_ref, b12_ref, z1_ref, zn1_ref, i - 2 * NB)

    @pl.when((3 * NB <= i) & (i < 4 * NB))
    def _():
        gcn_out(adj2_ref, s22_ref, b22_ref, z2_ref, zn2_ref, i - 3 * NB)

    @pl.when(4 * NB <= i)
    def _():
        k = i - 4 * NB
        cos = lax.dot_general(
            zn1_ref[pl.ds(k * BLK, BLK), :], zn2_ref[...],
            dimension_numbers=(((1,), (1,)), ((), ())),
            preferred_element_type=jnp.float32)
        sim = jnp.exp(cos * (1.0 / TAU))
        s = jnp.sum(sim, axis=1, keepdims=True)
        w = jnp.sum(sim * clm_ref[...], axis=1, keepdims=True)
        part = jnp.sum(jnp.log(s + 1e-8) - jnp.log(w))

        @pl.when(i == 4 * NB)
        def _():
            loss_ref[...] = jnp.zeros_like(loss_ref)

        loss_ref[...] += part


def _a1row(i):
    return jnp.where(i < NB, i,
           jnp.where(i < 2 * NB, NB - 1,
           jnp.where(i < 3 * NB, i - 2 * NB, NB - 1)))


def _a2row(i):
    return jnp.where(i < NB, 0,
           jnp.where(i < 2 * NB, i - NB,
           jnp.where(i < 3 * NB, NB - 1,
           jnp.where(i < 4 * NB, i - 3 * NB, NB - 1))))


def kernel(x1, adj1, x2, adj2, clm, W11, b11, W12, b12, W21, b21, W22, b22):
    bf = jnp.bfloat16
    z1, z2, acc = pl.pallas_call(
        _mega_body,
        grid=(5 * NB,),
        in_specs=[
            pl.BlockSpec((N, NFEAT), lambda i: (0, 0)),
            pl.BlockSpec((N, NFEAT), lambda i: (0, 0)),
            pl.BlockSpec((NFEAT, NFEAT), lambda i: (0, 0)),
            pl.BlockSpec((NFEAT, NFEAT), lambda i: (0, 0)),
            pl.BlockSpec((NFEAT, NHID), lambda i: (0, 0)),
            pl.BlockSpec((NFEAT, NHID), lambda i: (0, 0)),
            pl.BlockSpec((1, NFEAT), lambda i: (0, 0)),
            pl.BlockSpec((1, NFEAT), lambda i: (0, 0)),
            pl.BlockSpec((1, NHID), lambda i: (0, 0)),
            pl.BlockSpec((1, NHID), lambda i: (0, 0)),
            pl.BlockSpec((BLK, N), lambda i: (_a1row(i), 0)),
            pl.BlockSpec((BLK, N), lambda i: (_a2row(i), 0)),
            pl.BlockSpec((BLK, N), lambda i: (jnp.maximum(i - 4 * NB, 0), 0)),
        ],
        out_specs=(
            pl.BlockSpec((BLK, NHID), lambda i: (jnp.clip(i - 2 * NB, 0, NB - 1), 0)),
            pl.BlockSpec((BLK, NHID), lambda i: (jnp.clip(i - 3 * NB, 0, NB - 1), 0)),
            pl.BlockSpec((1, 1), lambda i: (0, 0)),
        ),
        out_shape=(
            jax.ShapeDtypeStruct((N, NHID), jnp.float32),
            jax.ShapeDtypeStruct((N, NHID), jnp.float32),
            jax.ShapeDtypeStruct((1, 1), jnp.float32),
        ),
        scratch_shapes=[
            pltpu.VMEM((N, NHID), bf),
            pltpu.VMEM((N, NHID), bf),
            pltpu.VMEM((N, NHID), bf),
            pltpu.VMEM((N, NHID), bf),
        ],
    )(x1.astype(bf), x2.astype(bf), W11.astype(bf), W21.astype(bf),
      W12.astype(bf), W22.astype(bf),
      b11.reshape(1, -1), b21.reshape(1, -1),
      b12.reshape(1, -1), b22.reshape(1, -1),
      adj1, adj2, clm)
    cl_loss = (acc[0, 0] / N).astype(jnp.float32).reshape(())
    return (z1, z2, cl_loss)


# speedup vs baseline: 1.0617x; 1.0617x over previous
import jax
import jax.numpy as jnp
from jax import lax
from jax.experimental import pallas as pl

N = 4096
NFEAT = 256
NHID = 128
TAU = 0.5
MBLK = 1024
OBLK = 1024
LBLK = 512


def _mid_body(adj_ref, x_ref, w1_ref, b1_ref, w2_ref, o_ref):
    t = jnp.dot(adj_ref[...].astype(jnp.bfloat16), x_ref[...],
                preferred_element_type=jnp.float32)
    t = jnp.dot(t.astype(jnp.bfloat16), w1_ref[...],
                preferred_element_type=jnp.float32)
    h = jnp.maximum(t + b1_ref[...], 0.0).astype(jnp.bfloat16)
    o_ref[...] = jnp.dot(h, w2_ref[...],
                         preferred_element_type=jnp.float32).astype(jnp.bfloat16)


def _out_body(adj_ref, s2_ref, b2_ref, z_ref, zn_ref):
    z = jnp.dot(adj_ref[...].astype(jnp.bfloat16), s2_ref[...],
                preferred_element_type=jnp.float32) + b2_ref[...]
    z_ref[...] = z
    nrm = jnp.sqrt(jnp.sum(z * z, axis=1, keepdims=True))
    zn_ref[...] = (z / nrm).astype(jnp.bfloat16)


def _loss_body(z1n_ref, z2n_ref, clm_ref, acc_ref):
    cos = lax.dot_general(
        z1n_ref[...], z2n_ref[...],
        dimension_numbers=(((1,), (1,)), ((), ())),
        preferred_element_type=jnp.float32)
    sim = jnp.exp(cos * (1.0 / TAU))
    s = jnp.sum(sim, axis=1, keepdims=True)
    w = jnp.sum(sim * clm_ref[...], axis=1, keepdims=True)
    part = jnp.sum(jnp.log(s + 1e-8) - jnp.log(w))

    @pl.when(pl.program_id(0) == 0)
    def _():
        acc_ref[...] = jnp.zeros_like(acc_ref)

    acc_ref[...] = acc_ref[...] + part


def _mid(adj, x_bf, W1, b1, W2):
    return pl.pallas_call(
        _mid_body,
        grid=(N // MBLK,),
        in_specs=[
            pl.BlockSpec((MBLK, N), lambda i: (i, 0)),
            pl.BlockSpec((N, NFEAT), lambda i: (0, 0)),
            pl.BlockSpec((NFEAT, NFEAT), lambda i: (0, 0)),
            pl.BlockSpec((1, NFEAT), lambda i: (0, 0)),
            pl.BlockSpec((NFEAT, NHID), lambda i: (0, 0)),
        ],
        out_specs=pl.BlockSpec((MBLK, NHID), lambda i: (i, 0)),
        out_shape=jax.ShapeDtypeStruct((N, NHID), jnp.bfloat16),
    )(adj, x_bf, W1, b1, W2)


def _outz(adj, s2, b2):
    return pl.pallas_call(
        _out_body,
        grid=(N // OBLK,),
        in_specs=[
            pl.BlockSpec((OBLK, N), lambda i: (i, 0)),
            pl.BlockSpec((N, NHID), lambda i: (0, 0)),
            pl.BlockSpec((1, NHID), lambda i: (0, 0)),
        ],
        out_specs=(
            pl.BlockSpec((OBLK, NHID), lambda i: (i, 0)),
            pl.BlockSpec((OBLK, NHID), lambda i: (i, 0)),
        ),
        out_shape=(
            jax.ShapeDtypeStruct((N, NHID), jnp.float32),
            jax.ShapeDtypeStruct((N, NHID), jnp.bfloat16),
        ),
    )(adj, s2, b2)


def _loss(zn1, zn2, clm):
    return pl.pallas_call(
        _loss_body,
        grid=(N // LBLK,),
        in_specs=[
            pl.BlockSpec((LBLK, NHID), lambda i: (i, 0)),
            pl.BlockSpec((N, NHID), lambda i: (0, 0)),
            pl.BlockSpec((LBLK, N), lambda i: (i, 0)),
        ],
        out_specs=pl.BlockSpec((1, 1), lambda i: (0, 0)),
        out_shape=jax.ShapeDtypeStruct((1, 1), jnp.float32),
    )(zn1, zn2, clm)


def kernel(x1, adj1, x2, adj2, clm, W11, b11, W12, b12, W21, b21, W22, b22):
    bf = jnp.bfloat16
    s21 = _mid(adj1, x1.astype(bf), W11.astype(bf), b11.reshape(1, -1),
               W12.astype(bf))
    s22 = _mid(adj2, x2.astype(bf), W21.astype(bf), b21.reshape(1, -1),
               W22.astype(bf))
    z1, zn1 = _outz(adj1, s21, b12.reshape(1, -1))
    z2, zn2 = _outz(adj2, s22, b22.reshape(1, -1))
    acc = _loss(zn1, zn2, clm)
    cl_loss = (acc[0, 0] / N).astype(jnp.float32).reshape(())
    return (z1, z2, cl_loss)


# speedup vs baseline: 1.1024x; 1.0383x over previous
import jax
import jax.numpy as jnp
from jax import lax
from jax.experimental import pallas as pl

N = 4096
NFEAT = 256
NHID = 128
TAU = 0.5
BLK = 512
NB = N // BLK
LBLK = 512


def _mid_body(adj1_ref, adj2_ref, x1_ref, x2_ref, w11_ref, w21_ref,
              b11_ref, b21_ref, w12_ref, w22_ref, s21_ref, s22_ref):
    i = pl.program_id(0)

    def tower(adj_ref, x_ref, w1_ref, b1_ref, w2_ref, o_ref):
        t = jnp.dot(adj_ref[...].astype(jnp.bfloat16), x_ref[...],
                    preferred_element_type=jnp.float32)
        t = jnp.dot(t.astype(jnp.bfloat16), w1_ref[...],
                    preferred_element_type=jnp.float32)
        h = jnp.maximum(t + b1_ref[...], 0.0).astype(jnp.bfloat16)
        o_ref[...] = jnp.dot(h, w2_ref[...],
                             preferred_element_type=jnp.float32).astype(jnp.bfloat16)

    @pl.when(i < NB)
    def _():
        tower(adj1_ref, x1_ref, w11_ref, b11_ref, w12_ref, s21_ref)

    @pl.when(i >= NB)
    def _():
        tower(adj2_ref, x2_ref, w21_ref, b21_ref, w22_ref, s22_ref)


def _out_body(adj1_ref, adj2_ref, s21_ref, s22_ref, b12_ref, b22_ref,
              z1_ref, z2_ref, zn1_ref, zn2_ref):
    i = pl.program_id(0)

    def tower(adj_ref, s2_ref, b2_ref, z_ref, zn_ref):
        z = jnp.dot(adj_ref[...].astype(jnp.bfloat16), s2_ref[...],
                    preferred_element_type=jnp.float32) + b2_ref[...]
        z_ref[...] = z
        nrm = jnp.sqrt(jnp.sum(z * z, axis=1, keepdims=True))
        zn_ref[...] = (z / nrm).astype(jnp.bfloat16)

    @pl.when(i < NB)
    def _():
        tower(adj1_ref, s21_ref, b12_ref, z1_ref, zn1_ref)

    @pl.when(i >= NB)
    def _():
        tower(adj2_ref, s22_ref, b22_ref, z2_ref, zn2_ref)


def _loss_body(z1n_ref, z2n_ref, clm_ref, acc_ref):
    cos = lax.dot_general(
        z1n_ref[...], z2n_ref[...],
        dimension_numbers=(((1,), (1,)), ((), ())),
        preferred_element_type=jnp.float32)
    sim = jnp.exp(cos * (1.0 / TAU))
    s = jnp.sum(sim, axis=1, keepdims=True)
    w = jnp.sum(sim * clm_ref[...], axis=1, keepdims=True)
    part = jnp.sum(jnp.log(s + 1e-8) - jnp.log(w))

    @pl.when(pl.program_id(0) == 0)
    def _():
        acc_ref[...] = jnp.zeros_like(acc_ref)

    acc_ref[...] = acc_ref[...] + part


def _r1(i):
    return jnp.minimum(i, NB - 1)


def _r2(i):
    return jnp.maximum(i - NB, 0)


def _mid(adj1, adj2, x1, x2, W11, W21, b11, b21, W12, W22):
    return pl.pallas_call(
        _mid_body,
        grid=(2 * NB,),
        in_specs=[
            pl.BlockSpec((BLK, N), lambda i: (_r1(i), 0)),
            pl.BlockSpec((BLK, N), lambda i: (_r2(i), 0)),
            pl.BlockSpec((N, NFEAT), lambda i: (0, 0)),
            pl.BlockSpec((N, NFEAT), lambda i: (0, 0)),
            pl.BlockSpec((NFEAT, NFEAT), lambda i: (0, 0)),
            pl.BlockSpec((NFEAT, NFEAT), lambda i: (0, 0)),
            pl.BlockSpec((1, NFEAT), lambda i: (0, 0)),
            pl.BlockSpec((1, NFEAT), lambda i: (0, 0)),
            pl.BlockSpec((NFEAT, NHID), lambda i: (0, 0)),
            pl.BlockSpec((NFEAT, NHID), lambda i: (0, 0)),
        ],
        out_specs=(
            pl.BlockSpec((BLK, NHID), lambda i: (_r1(i), 0)),
            pl.BlockSpec((BLK, NHID), lambda i: (_r2(i), 0)),
        ),
        out_shape=(
            jax.ShapeDtypeStruct((N, NHID), jnp.bfloat16),
            jax.ShapeDtypeStruct((N, NHID), jnp.bfloat16),
        ),
    )(adj1, adj2, x1, x2, W11, W21, b11, b21, W12, W22)


def _out(adj1, adj2, s21, s22, b12, b22):
    return pl.pallas_call(
        _out_body,
        grid=(2 * NB,),
        in_specs=[
            pl.BlockSpec((BLK, N), lambda i: (_r1(i), 0)),
            pl.BlockSpec((BLK, N), lambda i: (_r2(i), 0)),
            pl.BlockSpec((N, NHID), lambda i: (0, 0)),
            pl.BlockSpec((N, NHID), lambda i: (0, 0)),
            pl.BlockSpec((1, NHID), lambda i: (0, 0)),
            pl.BlockSpec((1, NHID), lambda i: (0, 0)),
        ],
        out_specs=(
            pl.BlockSpec((BLK, NHID), lambda i: (_r1(i), 0)),
            pl.BlockSpec((BLK, NHID), lambda i: (_r2(i), 0)),
            pl.BlockSpec((BLK, NHID), lambda i: (_r1(i), 0)),
            pl.BlockSpec((BLK, NHID), lambda i: (_r2(i), 0)),
        ),
        out_shape=(
            jax.ShapeDtypeStruct((N, NHID), jnp.float32),
            jax.ShapeDtypeStruct((N, NHID), jnp.float32),
            jax.ShapeDtypeStruct((N, NHID), jnp.bfloat16),
            jax.ShapeDtypeStruct((N, NHID), jnp.bfloat16),
        ),
    )(adj1, adj2, s21, s22, b12, b22)


def _loss(zn1, zn2, clm):
    return pl.pallas_call(
        _loss_body,
        grid=(N // LBLK,),
        in_specs=[
            pl.BlockSpec((LBLK, NHID), lambda i: (i, 0)),
            pl.BlockSpec((N, NHID), lambda i: (0, 0)),
            pl.BlockSpec((LBLK, N), lambda i: (i, 0)),
        ],
        out_specs=pl.BlockSpec((1, 1), lambda i: (0, 0)),
        out_shape=jax.ShapeDtypeStruct((1, 1), jnp.float32),
    )(zn1, zn2, clm)


def kernel(x1, adj1, x2, adj2, clm, W11, b11, W12, b12, W21, b21, W22, b22):
    bf = jnp.bfloat16
    s21, s22 = _mid(adj1, adj2, x1.astype(bf), x2.astype(bf),
                    W11.astype(bf), W21.astype(bf),
                    b11.reshape(1, -1), b21.reshape(1, -1),
                    W12.astype(bf), W22.astype(bf))
    z1, z2, zn1, zn2 = _out(adj1, adj2, s21, s22,
                            b12.reshape(1, -1), b22.reshape(1, -1))
    acc = _loss(zn1, zn2, clm)
    cl_loss = (acc[0, 0] / N).astype(jnp.float32).reshape(())
    return (z1, z2, cl_loss)


# speedup vs baseline: 1.1175x; 1.0137x over previous
import jax
import jax.numpy as jnp
from jax import lax
from jax.experimental import pallas as pl

N = 4096
NFEAT = 256
NHID = 128
TAU = 0.5
BLK = 512
LBLK = 512


def _mid_body(adj1_ref, adj2_ref, x1_ref, x2_ref, w11_ref, w21_ref,
              b11_ref, b21_ref, w12_ref, w22_ref, s21_ref, s22_ref):
    def tower(adj_ref, x_ref, w1_ref, b1_ref, w2_ref, o_ref):
        t = jnp.dot(adj_ref[...].astype(jnp.bfloat16), x_ref[...],
                    preferred_element_type=jnp.float32)
        t = jnp.dot(t.astype(jnp.bfloat16), w1_ref[...],
                    preferred_element_type=jnp.float32)
        h = jnp.maximum(t + b1_ref[...], 0.0).astype(jnp.bfloat16)
        o_ref[...] = jnp.dot(h, w2_ref[...],
                             preferred_element_type=jnp.float32).astype(jnp.bfloat16)

    tower(adj1_ref, x1_ref, w11_ref, b11_ref, w12_ref, s21_ref)
    tower(adj2_ref, x2_ref, w21_ref, b21_ref, w22_ref, s22_ref)


def _out_body(adj1_ref, adj2_ref, s21_ref, s22_ref, b12_ref, b22_ref,
              z1_ref, z2_ref, zn1_ref, zn2_ref):
    def tower(adj_ref, s2_ref, b2_ref, z_ref, zn_ref):
        z = jnp.dot(adj_ref[...].astype(jnp.bfloat16), s2_ref[...],
                    preferred_element_type=jnp.float32) + b2_ref[...]
        z_ref[...] = z
        nrm = jnp.sqrt(jnp.sum(z * z, axis=1, keepdims=True))
        zn_ref[...] = (z / nrm).astype(jnp.bfloat16)

    tower(adj1_ref, s21_ref, b12_ref, z1_ref, zn1_ref)
    tower(adj2_ref, s22_ref, b22_ref, z2_ref, zn2_ref)


def _loss_body(z1n_ref, z2n_ref, clml_ref, clmr_ref, acc_ref):
    cos = lax.dot_general(
        z1n_ref[...], z2n_ref[...],
        dimension_numbers=(((1,), (1,)), ((), ())),
        preferred_element_type=jnp.float32)
    sim = jnp.exp(cos * (1.0 / TAU))
    s = jnp.sum(sim, axis=1, keepdims=True)
    H = N // 2
    w = (jnp.sum(sim[:, :H] * clml_ref[...], axis=1, keepdims=True)
         + jnp.sum(sim[:, H:] * clmr_ref[...], axis=1, keepdims=True))
    part = jnp.sum(jnp.log(s + 1e-8) - jnp.log(w))

    @pl.when(pl.program_id(0) == 0)
    def _():
        acc_ref[...] = jnp.zeros_like(acc_ref)

    acc_ref[...] = acc_ref[...] + part


def _mid(adj1, adj2, x1, x2, W11, W21, b11, b21, W12, W22):
    return pl.pallas_call(
        _mid_body,
        grid=(N // BLK,),
        in_specs=[
            pl.BlockSpec((BLK, N), lambda i: (i, 0)),
            pl.BlockSpec((BLK, N), lambda i: (i, 0)),
            pl.BlockSpec((N, NFEAT), lambda i: (0, 0)),
            pl.BlockSpec((N, NFEAT), lambda i: (0, 0)),
            pl.BlockSpec((NFEAT, NFEAT), lambda i: (0, 0)),
            pl.BlockSpec((NFEAT, NFEAT), lambda i: (0, 0)),
            pl.BlockSpec((1, NFEAT), lambda i: (0, 0)),
            pl.BlockSpec((1, NFEAT), lambda i: (0, 0)),
            pl.BlockSpec((NFEAT, NHID), lambda i: (0, 0)),
            pl.BlockSpec((NFEAT, NHID), lambda i: (0, 0)),
        ],
        out_specs=(
            pl.BlockSpec((BLK, NHID), lambda i: (i, 0)),
            pl.BlockSpec((BLK, NHID), lambda i: (i, 0)),
        ),
        out_shape=(
            jax.ShapeDtypeStruct((N, NHID), jnp.bfloat16),
            jax.ShapeDtypeStruct((N, NHID), jnp.bfloat16),
        ),
    )(adj1, adj2, x1, x2, W11, W21, b11, b21, W12, W22)


def _out(adj1, adj2, s21, s22, b12, b22):
    return pl.pallas_call(
        _out_body,
        grid=(N // BLK,),
        in_specs=[
            pl.BlockSpec((BLK, N), lambda i: (i, 0)),
            pl.BlockSpec((BLK, N), lambda i: (i, 0)),
            pl.BlockSpec((N, NHID), lambda i: (0, 0)),
            pl.BlockSpec((N, NHID), lambda i: (0, 0)),
            pl.BlockSpec((1, NHID), lambda i: (0, 0)),
            pl.BlockSpec((1, NHID), lambda i: (0, 0)),
        ],
        out_specs=(
            pl.BlockSpec((BLK, NHID), lambda i: (i, 0)),
            pl.BlockSpec((BLK, NHID), lambda i: (i, 0)),
            pl.BlockSpec((BLK, NHID), lambda i: (i, 0)),
            pl.BlockSpec((BLK, NHID), lambda i: (i, 0)),
        ),
        out_shape=(
            jax.ShapeDtypeStruct((N, NHID), jnp.float32),
            jax.ShapeDtypeStruct((N, NHID), jnp.float32),
            jax.ShapeDtypeStruct((N, NHID), jnp.bfloat16),
            jax.ShapeDtypeStruct((N, NHID), jnp.bfloat16),
        ),
    )(adj1, adj2, s21, s22, b12, b22)


def _loss(zn1, zn2, clm):
    return pl.pallas_call(
        _loss_body,
        grid=(N // LBLK,),
        in_specs=[
            pl.BlockSpec((LBLK, NHID), lambda i: (i, 0)),
            pl.BlockSpec((N, NHID), lambda i: (0, 0)),
            pl.BlockSpec((LBLK, N // 2), lambda i: (i, 0)),
            pl.BlockSpec((LBLK, N // 2), lambda i: (i, 1)),
        ],
        out_specs=pl.BlockSpec((1, 1), lambda i: (0, 0)),
        out_shape=jax.ShapeDtypeStruct((1, 1), jnp.float32),
    )(zn1, zn2, clm, clm)


def kernel(x1, adj1, x2, adj2, clm, W11, b11, W12, b12, W21, b21, W22, b22):
    bf = jnp.bfloat16
    s21, s22 = _mid(adj1, adj2, x1.astype(bf), x2.astype(bf),
                    W11.astype(bf), W21.astype(bf),
                    b11.reshape(1, -1), b21.reshape(1, -1),
                    W12.astype(bf), W22.astype(bf))
    z1, z2, zn1, zn2 = _out(adj1, adj2, s21, s22,
                            b12.reshape(1, -1), b22.reshape(1, -1))
    acc = _loss(zn1, zn2, clm)
    cl_loss = (acc[0, 0] / N).astype(jnp.float32).reshape(())
    return (z1, z2, cl_loss)


# speedup vs baseline: 1.3246x; 1.1853x over previous
import jax
import jax.numpy as jnp
from jax import lax
from jax.experimental import pallas as pl
from jax.experimental.pallas import tpu as pltpu

N = 4096
NFEAT = 256
NHID = 128
TAU = 0.5
BLK = 512
NB = N // BLK
LBLK = 512


def _tower_body(adj_ref, x_ref, w1_ref, b1_ref, w2_ref, b2_ref,
                z_ref, zn_ref, adjv_ref, s2_ref):
    i = pl.program_id(0)

    @pl.when(i < NB)
    def _():
        a = adj_ref[...].astype(jnp.bfloat16)
        t = jnp.dot(a, x_ref[...], preferred_element_type=jnp.float32)
        t = jnp.dot(t.astype(jnp.bfloat16), w1_ref[...],
                    preferred_element_type=jnp.float32)
        h = jnp.maximum(t + b1_ref[...], 0.0).astype(jnp.bfloat16)
        s2_ref[pl.ds(i * BLK, BLK), :] = jnp.dot(
            h, w2_ref[...], preferred_element_type=jnp.float32
        ).astype(jnp.bfloat16)
        adjv_ref[pl.ds(i * BLK, BLK), :] = a

    @pl.when(i >= NB)
    def _():
        k = i - NB
        z = jnp.dot(adjv_ref[pl.ds(k * BLK, BLK), :], s2_ref[...],
                    preferred_element_type=jnp.float32) + b2_ref[...]
        z_ref[...] = z
        nrm = jnp.sqrt(jnp.sum(z * z, axis=1, keepdims=True))
        zn_ref[...] = (z / nrm).astype(jnp.bfloat16)


def _loss_body(z1n_ref, z2n_ref, clm_ref, acc_ref):
    cos = lax.dot_general(
        z1n_ref[...], z2n_ref[...],
        dimension_numbers=(((1,), (1,)), ((), ())),
        preferred_element_type=jnp.float32)
    sim = jnp.exp(cos * (1.0 / TAU))
    s = jnp.sum(sim, axis=1, keepdims=True)
    w = jnp.sum(sim * clm_ref[...], axis=1, keepdims=True)
    part = jnp.sum(jnp.log(s + 1e-8) - jnp.log(w))

    @pl.when(pl.program_id(0) == 0)
    def _():
        acc_ref[...] = jnp.zeros_like(acc_ref)

    acc_ref[...] = acc_ref[...] + part


def _tower(adj, x_bf, W1, b1, W2, b2):
    return pl.pallas_call(
        _tower_body,
        grid=(2 * NB,),
        in_specs=[
            pl.BlockSpec((BLK, N), lambda i: (jnp.minimum(i, NB - 1), 0)),
            pl.BlockSpec((N, NFEAT), lambda i: (0, 0)),
            pl.BlockSpec((NFEAT, NFEAT), lambda i: (0, 0)),
            pl.BlockSpec((1, NFEAT), lambda i: (0, 0)),
            pl.BlockSpec((NFEAT, NHID), lambda i: (0, 0)),
            pl.BlockSpec((1, NHID), lambda i: (0, 0)),
        ],
        out_specs=(
            pl.BlockSpec((BLK, NHID), lambda i: (jnp.maximum(i - NB, 0), 0)),
            pl.BlockSpec((BLK, NHID), lambda i: (jnp.maximum(i - NB, 0), 0)),
        ),
        out_shape=(
            jax.ShapeDtypeStruct((N, NHID), jnp.float32),
            jax.ShapeDtypeStruct((N, NHID), jnp.bfloat16),
        ),
        scratch_shapes=[
            pltpu.VMEM((N, N), jnp.bfloat16),
            pltpu.VMEM((N, NHID), jnp.bfloat16),
        ],
    )(adj, x_bf, W1, b1, W2, b2)


def _loss(zn1, zn2, clm):
    return pl.pallas_call(
        _loss_body,
        grid=(N // LBLK,),
        in_specs=[
            pl.BlockSpec((LBLK, NHID), lambda i: (i, 0)),
            pl.BlockSpec((N, NHID), lambda i: (0, 0)),
            pl.BlockSpec((LBLK, N), lambda i: (i, 0)),
        ],
        out_specs=pl.BlockSpec((1, 1), lambda i: (0, 0)),
        out_shape=jax.ShapeDtypeStruct((1, 1), jnp.float32),
    )(zn1, zn2, clm)


def kernel(x1, adj1, x2, adj2, clm, W11, b11, W12, b12, W21, b21, W22, b22):
    bf = jnp.bfloat16
    z1, zn1 = _tower(adj1, x1.astype(bf), W11.astype(bf), b11.reshape(1, -1),
                     W12.astype(bf), b12.reshape(1, -1))
    z2, zn2 = _tower(adj2, x2.astype(bf), W21.astype(bf), b21.reshape(1, -1),
                     W22.astype(bf), b22.reshape(1, -1))
    acc = _loss(zn1, zn2, clm)
    cl_loss = (acc[0, 0] / N).astype(jnp.float32).reshape(())
    return (z1, z2, cl_loss)
